# baseline (speedup 1.0000x reference)
import functools

import jax
import jax.numpy as jnp
from jax import lax
from jax.experimental import pallas as pl
from jax.experimental.pallas import tpu as pltpu
from jax.experimental.pallas import tpu_sc as plsc

BATCH = 4096
SEQ = 50
EMB = 128
NUM_CLASSES = 5

NC = 2
NS = 16
NW = NC * NS
BPW = BATCH // NW
LANES = 16
NJ = EMB // LANES


def _pool_body(x_hbm, table_hbm, out_hbm, idx_v, buf_v, acc_v, sem0, sem1):
    wid = lax.axis_index("s") * NC + lax.axis_index("c")
    base = wid * BPW

    pltpu.sync_copy(x_hbm.at[pl.ds(base, BPW)], idx_v)

    sems = (sem0, sem1)

    def issue(c, par):
        return pltpu.async_copy(table_hbm.at[idx_v.at[c]], buf_v.at[par],
                                sems[par])

    issue(0, 0)
    issue(1, 1)

    def accumulate(par, c):
        zero = tuple(jnp.zeros((LANES,), jnp.float32) for _ in range(NJ))

        @plsc.parallel_loop(0, SEQ, unroll=5, carry=zero)
        def accs(s, accs):
            return tuple(
                accs[j] + buf_v[par, s, pl.ds(j * LANES, LANES)]
                for j in range(NJ))

        for j in range(NJ):
            acc_v[c, pl.ds(j * LANES, LANES)] = accs[j]

    def outer(g, _):
        for par in range(2):
            c = g * 2 + par
            pltpu.make_async_copy(table_hbm.at[idx_v.at[0]], buf_v.at[par],
                                  sems[par]).wait()
            accumulate(par, c)

            @pl.when(c + 2 < BPW)
            def _():
                issue(c + 2, par)
        return 0

    lax.fori_loop(0, BPW // 2, outer, 0)

    pltpu.sync_copy(acc_v, out_hbm.at[pl.ds(base, BPW)])


_POOL_CACHE = []


def _pool(xi, table):
    if not _POOL_CACHE:
        _POOL_CACHE.append(functools.partial(
            pl.kernel,
            out_type=jax.ShapeDtypeStruct((BATCH, EMB), jnp.float32),
            mesh=plsc.VectorSubcoreMesh(core_axis_name="c",
                                        subcore_axis_name="s"),
            scratch_types=[
                pltpu.VMEM((BPW, SEQ), jnp.int32),
                pltpu.VMEM((2, SEQ, EMB), jnp.float32),
                pltpu.VMEM((BPW, EMB), jnp.float32),
                pltpu.SemaphoreType.DMA,
                pltpu.SemaphoreType.DMA,
            ],
        )(_pool_body))
    return _POOL_CACHE[0](xi, table)


def _cls_body(h_ref, w_ref, b_ref, o_ref):
    h = jnp.maximum(h_ref[...] * (1.0 / SEQ), 0.0)
    logits = lax.dot_general(h, w_ref[...], (((1,), (1,)), ((), ())),
                             preferred_element_type=jnp.float32)
    logits = logits + b_ref[...]
    m = jnp.max(logits, axis=1, keepdims=True)
    ex = jnp.exp(logits - m)
    lse = jnp.log(jnp.sum(ex, axis=1, keepdims=True)) + m
    o_ref[...] = logits - lse


def _classifier(pooled, W, b):
    blk = 1024
    grid = BATCH // blk
    return pl.pallas_call(
        _cls_body,
        grid=(grid,),
        in_specs=[
            pl.BlockSpec((blk, EMB), lambda i: (i, 0)),
            pl.BlockSpec((NUM_CLASSES, EMB), lambda i: (0, 0)),
            pl.BlockSpec((1, NUM_CLASSES), lambda i: (0, 0)),
        ],
        out_specs=pl.BlockSpec((blk, NUM_CLASSES), lambda i: (i, 0)),
        out_shape=jax.ShapeDtypeStruct((BATCH, NUM_CLASSES), jnp.float32),
    )(pooled, W, b)


def kernel(x, table, W, b):
    pooled = _pool(x.astype(jnp.int32), table)
    return _classifier(pooled, W, b.reshape(1, NUM_CLASSES))


# speedup vs baseline: 1.4295x; 1.4295x over previous
import functools

import jax
import jax.numpy as jnp
from jax import lax
from jax.experimental import pallas as pl
from jax.experimental.pallas import tpu as pltpu
from jax.experimental.pallas import tpu_sc as plsc

BATCH = 4096
SEQ = 50
EMB = 128
NUM_CLASSES = 5

NC = 2
NS = 16
NW = NC * NS
BPW = BATCH // NW
NBUF = 4
LANES = 16
NJ = EMB // LANES


def _pool_body(x_hbm, table_hbm, out_hbm, idx_v, buf_v, acc_v,
               sem0, sem1, sem2, sem3):
    wid = lax.axis_index("s") * NC + lax.axis_index("c")
    base = wid * BPW

    pltpu.sync_copy(x_hbm.at[pl.ds(base, BPW)], idx_v)

    sems = (sem0, sem1, sem2, sem3)

    def issue(c, par):
        return pltpu.async_copy(table_hbm.at[idx_v.at[c]], buf_v.at[par],
                                sems[par])

    for p0 in range(NBUF):
        issue(p0, p0)

    def accumulate(par, c):
        zero = tuple(jnp.zeros((LANES,), jnp.float32) for _ in range(NJ))

        @plsc.parallel_loop(0, SEQ, unroll=5, carry=zero)
        def accs(s, accs):
            return tuple(
                accs[j] + buf_v[par, s, pl.ds(j * LANES, LANES)]
                for j in range(NJ))

        for j in range(NJ):
            acc_v[c, pl.ds(j * LANES, LANES)] = accs[j]

    def outer(g, _):
        for par in range(NBUF):
            c = g * NBUF + par
            pltpu.make_async_copy(table_hbm.at[idx_v.at[0]], buf_v.at[par],
                                  sems[par]).wait()
            accumulate(par, c)

            @pl.when(c + NBUF < BPW)
            def _():
                issue(c + NBUF, par)
        return 0

    lax.fori_loop(0, BPW // NBUF, outer, 0)

    pltpu.sync_copy(acc_v, out_hbm.at[pl.ds(base, BPW)])


_POOL_CACHE = []


def _pool(xi, table):
    if not _POOL_CACHE:
        _POOL_CACHE.append(functools.partial(
            pl.kernel,
            out_type=jax.ShapeDtypeStruct((BATCH, EMB), jnp.float32),
            mesh=plsc.VectorSubcoreMesh(core_axis_name="c",
                                        subcore_axis_name="s"),
            scratch_types=[
                pltpu.VMEM((BPW, SEQ), jnp.int32),
                pltpu.VMEM((NBUF, SEQ, EMB), jnp.float32),
                pltpu.VMEM((BPW, EMB), jnp.float32),
                pltpu.SemaphoreType.DMA,
                pltpu.SemaphoreType.DMA,
                pltpu.SemaphoreType.DMA,
                pltpu.SemaphoreType.DMA,
            ],
        )(_pool_body))
    return _POOL_CACHE[0](xi, table)


def _cls_body(h_ref, w_ref, b_ref, o_ref):
    h = jnp.maximum(h_ref[...] * (1.0 / SEQ), 0.0)
    logits = lax.dot_general(h, w_ref[...], (((1,), (1,)), ((), ())),
                             preferred_element_type=jnp.float32)
    logits = logits + b_ref[...]
    m = jnp.max(logits, axis=1, keepdims=True)
    ex = jnp.exp(logits - m)
    lse = jnp.log(jnp.sum(ex, axis=1, keepdims=True)) + m
    o_ref[...] = logits - lse


def _classifier(pooled, W, b):
    blk = 1024
    grid = BATCH // blk
    return pl.pallas_call(
        _cls_body,
        grid=(grid,),
        in_specs=[
            pl.BlockSpec((blk, EMB), lambda i: (i, 0)),
            pl.BlockSpec((NUM_CLASSES, EMB), lambda i: (0, 0)),
            pl.BlockSpec((1, NUM_CLASSES), lambda i: (0, 0)),
        ],
        out_specs=pl.BlockSpec((blk, NUM_CLASSES), lambda i: (i, 0)),
        out_shape=jax.ShapeDtypeStruct((BATCH, NUM_CLASSES), jnp.float32),
    )(pooled, W, b)


def kernel(x, table, W, b):
    pooled = _pool(x.astype(jnp.int32), table)
    return _classifier(pooled, W, b.reshape(1, NUM_CLASSES))


# speedup vs baseline: 1.6262x; 1.1376x over previous
import functools

import jax
import jax.numpy as jnp
from jax import lax
from jax.experimental import pallas as pl
from jax.experimental.pallas import tpu as pltpu
from jax.experimental.pallas import tpu_sc as plsc

BATCH = 4096
SEQ = 50
EMB = 128
NUM_CLASSES = 5

NC = 2
NS = 16
NW = NC * NS
BPW = BATCH // NW
NBUF = 8
LANES = 16
NJ = EMB // LANES


def _pool_body(x_hbm, table_hbm, out_hbm, idx_v, buf_v, acc_v,
               sem0, sem1, sem2, sem3, sem4, sem5, sem6, sem7):
    wid = lax.axis_index("s") * NC + lax.axis_index("c")
    base = wid * BPW

    pltpu.sync_copy(x_hbm.at[pl.ds(base, BPW)], idx_v)

    sems = (sem0, sem1, sem2, sem3, sem4, sem5, sem6, sem7)

    def issue(c, par):
        return pltpu.async_copy(table_hbm.at[idx_v.at[c]], buf_v.at[par],
                                sems[par])

    for p0 in range(NBUF):
        issue(p0, p0)

    def accumulate(par, c):
        zero = tuple(jnp.zeros((LANES,), jnp.float32) for _ in range(NJ))

        @plsc.parallel_loop(0, SEQ, unroll=5, carry=zero)
        def accs(s, accs):
            return tuple(
                accs[j] + buf_v[par, s, pl.ds(j * LANES, LANES)]
                for j in range(NJ))

        for j in range(NJ):
            acc_v[c, pl.ds(j * LANES, LANES)] = accs[j]

    def outer(g, _):
        for par in range(NBUF):
            c = g * NBUF + par
            pltpu.make_async_copy(table_hbm.at[idx_v.at[0]], buf_v.at[par],
                                  sems[par]).wait()
            accumulate(par, c)

            @pl.when(c + NBUF < BPW)
            def _():
                issue(c + NBUF, par)
        return 0

    lax.fori_loop(0, BPW // NBUF, outer, 0)

    pltpu.sync_copy(acc_v, out_hbm.at[pl.ds(base, BPW)])


_POOL_CACHE = []


def _pool(xi, table):
    if not _POOL_CACHE:
        _POOL_CACHE.append(functools.partial(
            pl.kernel,
            out_type=jax.ShapeDtypeStruct((BATCH, EMB), jnp.float32),
            mesh=plsc.VectorSubcoreMesh(core_axis_name="c",
                                        subcore_axis_name="s"),
            scratch_types=[
                pltpu.VMEM((BPW, SEQ), jnp.int32),
                pltpu.VMEM((NBUF, SEQ, EMB), jnp.float32),
                pltpu.VMEM((BPW, EMB), jnp.float32),
            ] + [pltpu.SemaphoreType.DMA] * 8,
        )(_pool_body))
    return _POOL_CACHE[0](xi, table)


def _cls_body(h_ref, w_ref, b_ref, o_ref):
    h = jnp.maximum(h_ref[...] * (1.0 / SEQ), 0.0)
    logits = lax.dot_general(h, w_ref[...], (((1,), (1,)), ((), ())),
                             preferred_element_type=jnp.float32)
    logits = logits + b_ref[...]
    m = jnp.max(logits, axis=1, keepdims=True)
    ex = jnp.exp(logits - m)
    lse = jnp.log(jnp.sum(ex, axis=1, keepdims=True)) + m
    o_ref[...] = logits - lse


def _classifier(pooled, W, b):
    blk = 1024
    grid = BATCH // blk
    return pl.pallas_call(
        _cls_body,
        grid=(grid,),
        in_specs=[
            pl.BlockSpec((blk, EMB), lambda i: (i, 0)),
            pl.BlockSpec((NUM_CLASSES, EMB), lambda i: (0, 0)),
            pl.BlockSpec((1, NUM_CLASSES), lambda i: (0, 0)),
        ],
        out_specs=pl.BlockSpec((blk, NUM_CLASSES), lambda i: (i, 0)),
        out_shape=jax.ShapeDtypeStruct((BATCH, NUM_CLASSES), jnp.float32),
    )(pooled, W, b)


def kernel(x, table, W, b):
    pooled = _pool(x.astype(jnp.int32), table)
    return _classifier(pooled, W, b.reshape(1, NUM_CLASSES))
